# baseline (speedup 1.0000x reference)
import functools

import jax
import jax.numpy as jnp
from jax import lax
from jax.experimental import pallas as pl
from jax.experimental.pallas import tpu as pltpu
from jax.experimental.pallas import tpu_sc as plsc

N = 10000
E = 320000
D = 128
B = 64
NC, NS = 2, 16
NW = NC * NS
NPAD = 10240
CHUNK = 128
E_PAD = 327680
EXTRA = E_PAD - E
TCHK = E_PAD // (NW * CHUNK)
PIPE = 2
SEG = 5
SCH = TCHK // SEG
NGRP = SCH // PIPE
RPT = NPAD // NS
ZR = 16

_mesh = plsc.VectorSubcoreMesh(
    core_axis_name="c", subcore_axis_name="s", num_cores=NC, num_subcores=NS)


@functools.partial(
    pl.kernel,
    out_type=jax.ShapeDtypeStruct((NC, 2, NPAD), jnp.float32),
    mesh=_mesh,
    scratch_types=[
        pltpu.VMEM_SHARED((NPAD,), jnp.float32),
        pltpu.VMEM_SHARED((NPAD,), jnp.float32),
        pltpu.VMEM((CHUNK,), jnp.int32),
        pltpu.VMEM((CHUNK,), jnp.int32),
        pltpu.VMEM((CHUNK,), jnp.int32),
        pltpu.VMEM((CHUNK,), jnp.int32),
        pltpu.VMEM((CHUNK,), jnp.float32),
        pltpu.VMEM((RPT,), jnp.float32),
        pltpu.SemaphoreType.DMA,
        pltpu.SemaphoreType.DMA,
        pltpu.SemaphoreType.DMA,
        pltpu.SemaphoreType.DMA,
    ],
)
def _deg_kernel(src_hbm, dst_hbm, out_hbm, do_sp, di_sp, is0, is1, id0, id1,
                ones_v, zb_v, il0, il1, sc0, sc1):
    isb = (is0, is1)
    idb = (id0, id1)
    ilsems = (il0, il1)
    scsems = (sc0, sc1)
    c = lax.axis_index("c")
    s = lax.axis_index("s")
    w = c * NS + s

    def _fill(i, _):
        zb_v[pl.ds(i * 16, 16)] = jnp.zeros((16,), jnp.float32)
        return 0
    lax.fori_loop(0, RPT // 16, _fill, 0)

    def _fill1(i, _):
        ones_v[pl.ds(i * 16, 16)] = jnp.ones((16,), jnp.float32)
        return 0
    lax.fori_loop(0, CHUNK // 16, _fill1, 0)

    pltpu.sync_copy(zb_v, do_sp.at[pl.ds(s * RPT, RPT)])
    pltpu.sync_copy(zb_v, di_sp.at[pl.ds(s * RPT, RPT)])
    plsc.subcore_barrier()

    def _grp(t, _):
        for k in range(PIPE):
            @pl.when(t > 0)
            def _():
                pltpu.make_async_copy(
                    ones_v, do_sp.at[isb[k]], scsems[k]).wait()
                pltpu.make_async_copy(
                    ones_v, di_sp.at[idb[k]], scsems[k]).wait()
            row = w * TCHK + PIPE * t + k
            pltpu.async_copy(src_hbm.at[row], isb[k], ilsems[k])
            pltpu.async_copy(dst_hbm.at[row], idb[k], ilsems[k])
        for k in range(PIPE):
            row = w * TCHK + PIPE * t + k
            pltpu.make_async_copy(src_hbm.at[row], isb[k], ilsems[k]).wait()
            pltpu.make_async_copy(dst_hbm.at[row], idb[k], ilsems[k]).wait()
            pltpu.async_copy(ones_v, do_sp.at[isb[k]], scsems[k], add=True)
            pltpu.async_copy(ones_v, di_sp.at[idb[k]], scsems[k], add=True)
        return 0
    lax.fori_loop(0, TCHK // PIPE, _grp, 0)
    for k in range(PIPE):
        pltpu.make_async_copy(ones_v, do_sp.at[isb[k]], scsems[k]).wait()
        pltpu.make_async_copy(ones_v, di_sp.at[idb[k]], scsems[k]).wait()
    plsc.subcore_barrier()

    pltpu.sync_copy(do_sp.at[pl.ds(s * RPT, RPT)], zb_v)
    pltpu.sync_copy(zb_v, out_hbm.at[c, 0, pl.ds(s * RPT, RPT)])
    pltpu.sync_copy(di_sp.at[pl.ds(s * RPT, RPT)], zb_v)
    pltpu.sync_copy(zb_v, out_hbm.at[c, 1, pl.ds(s * RPT, RPT)])


@functools.partial(
    pl.kernel,
    out_type=jax.ShapeDtypeStruct((NC, NPAD, D), jnp.float32),
    mesh=_mesh,
    scratch_types=[
        pltpu.VMEM_SHARED((NPAD, D), jnp.float32),
        pltpu.VMEM((2, CHUNK), jnp.int32),
        pltpu.VMEM((2, CHUNK), jnp.int32),
        pltpu.VMEM((CHUNK,), jnp.int32),
        pltpu.VMEM((CHUNK,), jnp.int32),
        pltpu.VMEM((PIPE, CHUNK, D), jnp.float32),
        pltpu.VMEM((ZR, D), jnp.float32),
        pltpu.SemaphoreType.DMA,
        pltpu.SemaphoreType.DMA,
        pltpu.SemaphoreType.DMA,
        pltpu.SemaphoreType.DMA,
    ],
)
def _agg_kernel(hs_hbm, idx2_hbm, out_hbm, acc_sp, ib0, ib1, id0, id1,
                rows_v, zb_v, g0, g1, s0, s1):
    gsems = (g0, g1)
    ssems = (s0, s1)
    ibb = (ib0, ib1)
    idb = (id0, id1)
    c = lax.axis_index("c")
    s = lax.axis_index("s")
    w = c * NS + s

    def _zrow(i, _):
        for k in range(D // 16):
            zb_v[i, pl.ds(16 * k, 16)] = jnp.zeros((16,), jnp.float32)
        return 0
    lax.fori_loop(0, ZR, _zrow, 0)

    def _zacc(i, _):
        pltpu.sync_copy(zb_v, acc_sp.at[pl.ds(s * RPT + ZR * i, ZR)])
        return 0
    lax.fori_loop(0, RPT // ZR, _zacc, 0)
    plsc.subcore_barrier()

    def _grp(t, _):
        for k in range(PIPE):
            row = w * TCHK + PIPE * t + k

            @pl.when(t > 0)
            def _():
                pltpu.make_async_copy(
                    rows_v.at[k], acc_sp.at[idb[k]], ssems[k]).wait()
            pltpu.sync_copy(idx2_hbm.at[row], ibb[k])
            pltpu.async_copy(hs_hbm.at[ibb[k].at[0]], rows_v.at[k], gsems[k])
            for m in range(CHUNK // 16):
                idb[k][pl.ds(16 * m, 16)] = ibb[k][1, pl.ds(16 * m, 16)]
        for k in range(PIPE):
            pltpu.make_async_copy(
                hs_hbm.at[ibb[k].at[0]], rows_v.at[k], gsems[k]).wait()
            pltpu.async_copy(
                rows_v.at[k], acc_sp.at[idb[k]], ssems[k], add=True)
        return 0
    lax.fori_loop(0, TCHK // PIPE, _grp, 0)
    for k in range(PIPE):
        pltpu.make_async_copy(
            rows_v.at[k], acc_sp.at[idb[k]], ssems[k]).wait()
    plsc.subcore_barrier()

    pltpu.sync_copy(acc_sp.at[pl.ds(s * RPT, RPT)],
                    out_hbm.at[c, pl.ds(s * RPT, RPT)])


BR = 512
NBLK = NPAD // BR


def _tc1_body(x_ref, w_ref, d0_ref, d1_ref, o_ref):
    inv = lax.rsqrt(jnp.maximum(d0_ref[...] + d1_ref[...], 1.0))
    h = jnp.dot(x_ref[...], w_ref[...], preferred_element_type=jnp.float32)
    o_ref[...] = h * inv[:, None]


def _tc1(x, W, do0, do1):
    return pl.pallas_call(
        _tc1_body,
        grid=(NBLK,),
        in_specs=[
            pl.BlockSpec((BR, D), lambda i: (i, 0)),
            pl.BlockSpec((D, D), lambda i: (0, 0)),
            pl.BlockSpec((BR,), lambda i: (i,)),
            pl.BlockSpec((BR,), lambda i: (i,)),
        ],
        out_specs=pl.BlockSpec((BR, D), lambda i: (i, 0)),
        out_shape=jax.ShapeDtypeStruct((NPAD, D), jnp.float32),
    )(x, W, do0, do1)


def _tc2_body(a0_ref, a1_ref, di0_ref, di1_ref, do0_ref, do1_ref,
              b_ref, g_ref, be_ref, w_ref, o_ref):
    agg = a0_ref[...] + a1_ref[...]
    inv_in = lax.rsqrt(jnp.maximum(di0_ref[...] + di1_ref[...], 1.0))
    out1 = g_ref[...] * (agg * inv_in[:, None] + b_ref[...]) + be_ref[...]
    out1 = jnp.maximum(out1, 0.0)
    inv_out = lax.rsqrt(jnp.maximum(do0_ref[...] + do1_ref[...], 1.0))
    h = jnp.dot(out1, w_ref[...], preferred_element_type=jnp.float32)
    o_ref[...] = h * inv_out[:, None]


def _tc2(a0, a1, di0, di1, do0, do1, b, g, be, W):
    return pl.pallas_call(
        _tc2_body,
        grid=(NBLK,),
        in_specs=[
            pl.BlockSpec((BR, D), lambda i: (i, 0)),
            pl.BlockSpec((BR, D), lambda i: (i, 0)),
            pl.BlockSpec((BR,), lambda i: (i,)),
            pl.BlockSpec((BR,), lambda i: (i,)),
            pl.BlockSpec((BR,), lambda i: (i,)),
            pl.BlockSpec((BR,), lambda i: (i,)),
            pl.BlockSpec((D,), lambda i: (0,)),
            pl.BlockSpec((D,), lambda i: (0,)),
            pl.BlockSpec((D,), lambda i: (0,)),
            pl.BlockSpec((D, D), lambda i: (0, 0)),
        ],
        out_specs=pl.BlockSpec((BR, D), lambda i: (i, 0)),
        out_shape=jax.ShapeDtypeStruct((NPAD, D), jnp.float32),
    )(a0, a1, di0, di1, do0, do1, b, g, be, W)


def _tc3_body(a0_ref, a1_ref, di0_ref, di1_ref, gid_ref, b_ref, g_ref,
              be_ref, wf1_ref, bf1_ref, wf2_ref, bf2_ref, prot_ref,
              wo1_ref, bo1_ref, wo2_ref, bo2_ref, wo3_ref, bo3_ref,
              z_ref, comp_ref, acc_ref):
    i = pl.program_id(0)

    @pl.when(i == 0)
    def _():
        acc_ref[...] = jnp.full((B, D), -jnp.inf, jnp.float32)

    agg = a0_ref[...] + a1_ref[...]
    inv_in = lax.rsqrt(jnp.maximum(di0_ref[...] + di1_ref[...], 1.0))
    f = g_ref[...] * (agg * inv_in[:, None] + b_ref[...]) + be_ref[...]
    f = jnp.maximum(f, 0.0)

    rows = i * BR + lax.broadcasted_iota(jnp.int32, (BR,), 0)
    valid = rows < N
    gid = gid_ref[...]
    g_lo = jnp.min(jnp.where(valid, gid, B - 1))
    g_hi = jnp.max(jnp.where(valid, gid, 0))

    def _upd(b, _):
        m = (gid == b) & valid
        p = jnp.where(m, 0.0, -jnp.inf)
        v = jnp.max(f + p[:, None], axis=0)
        cur = acc_ref[pl.ds(b, 1), :]
        acc_ref[pl.ds(b, 1), :] = jnp.maximum(cur, v[None, :])
        return 0
    lax.fori_loop(g_lo, g_hi + 1, _upd, 0)

    @pl.when(i == pl.num_programs(0) - 1)
    def _():
        pooled = acc_ref[...]
        h = jnp.dot(pooled, wf1_ref[...], preferred_element_type=jnp.float32)
        h = jnp.maximum(h + bf1_ref[...], 0.0)
        comp = jnp.dot(h, wf2_ref[...],
                       preferred_element_type=jnp.float32) + bf2_ref[...]
        comp_ref[...] = comp
        h2 = (jnp.dot(comp, wo1_ref[0:D, :], preferred_element_type=jnp.float32)
              + jnp.dot(prot_ref[...], wo1_ref[D:2 * D, :],
                        preferred_element_type=jnp.float32) + bo1_ref[...])
        h2 = jnp.maximum(h2, 0.0)
        h3 = jnp.maximum(
            jnp.dot(h2, wo2_ref[...], preferred_element_type=jnp.float32)
            + bo2_ref[...], 0.0)
        z_ref[...] = jnp.dot(
            h3, wo3_ref[...], preferred_element_type=jnp.float32) + bo3_ref[...]


def _tc3(a0, a1, di0, di1, gid, b, g, be, Wf1, bf1, Wf2, bf2, prot,
         Wo1, bo1, Wo2, bo2, Wo3, bo3):
    H2 = 2 * 128
    return pl.pallas_call(
        _tc3_body,
        grid=(NBLK,),
        in_specs=[
            pl.BlockSpec((BR, D), lambda i: (i, 0)),
            pl.BlockSpec((BR, D), lambda i: (i, 0)),
            pl.BlockSpec((BR,), lambda i: (i,)),
            pl.BlockSpec((BR,), lambda i: (i,)),
            pl.BlockSpec((BR,), lambda i: (i,)),
            pl.BlockSpec((D,), lambda i: (0,)),
            pl.BlockSpec((D,), lambda i: (0,)),
            pl.BlockSpec((D,), lambda i: (0,)),
            pl.BlockSpec((D, H2), lambda i: (0, 0)),
            pl.BlockSpec((H2,), lambda i: (0,)),
            pl.BlockSpec((H2, D), lambda i: (0, 0)),
            pl.BlockSpec((D,), lambda i: (0,)),
            pl.BlockSpec((B, D), lambda i: (0, 0)),
            pl.BlockSpec((2 * D, H2), lambda i: (0, 0)),
            pl.BlockSpec((H2,), lambda i: (0,)),
            pl.BlockSpec((H2, D), lambda i: (0, 0)),
            pl.BlockSpec((D,), lambda i: (0,)),
            pl.BlockSpec((D, 2), lambda i: (0, 0)),
            pl.BlockSpec((2,), lambda i: (0,)),
        ],
        out_specs=[
            pl.BlockSpec((B, 2), lambda i: (0, 0)),
            pl.BlockSpec((B, D), lambda i: (0, 0)),
        ],
        out_shape=[
            jax.ShapeDtypeStruct((B, 2), jnp.float32),
            jax.ShapeDtypeStruct((B, D), jnp.float32),
        ],
        scratch_shapes=[pltpu.VMEM((B, D), jnp.float32)],
    )(a0, a1, di0, di1, gid, b, g, be, Wf1, bf1, Wf2, bf2, prot,
      Wo1, bo1, Wo2, bo2, Wo3, bo3)


def kernel(x, edge_index, graph_ids, prot_bat, prot_list,
           W_g1, b_g1, gamma1, beta1, W_g2, b_g2, gamma2, beta2,
           W_fc1g, b_fc1g, W_fc2g, b_fc2g,
           W_o1, b_o1, W_o2, b_o2, W_o3, b_o3):
    pad_idx = N + jnp.arange(EXTRA, dtype=jnp.int32) % (NPAD - N)
    src2 = jnp.concatenate([edge_index[0], pad_idx]).reshape(-1, CHUNK)
    dst2 = jnp.concatenate([edge_index[1], pad_idx]).reshape(-1, CHUNK)
    idx2 = jnp.stack([src2, dst2], axis=1)
    x_p = jnp.pad(x, ((0, NPAD - N), (0, 0)))

    degs = _deg_kernel(src2, dst2)
    do0, do1 = degs[0, 0], degs[1, 0]
    di0, di1 = degs[0, 1], degs[1, 1]

    hs1 = _tc1(x_p, W_g1, do0, do1)
    agg1 = _agg_kernel(hs1, idx2)
    hs2 = _tc2(agg1[0], agg1[1], di0, di1, do0, do1,
               b_g1, gamma1, beta1, W_g2)
    agg2 = _agg_kernel(hs2, idx2)
    z, comp = _tc3(agg2[0], agg2[1], di0, di1, graph_ids,
                   b_g2, gamma2, beta2,
                   W_fc1g, b_fc1g, W_fc2g, b_fc2g, prot_bat,
                   W_o1, b_o1, W_o2, b_o2, W_o3, b_o3)
    return (z, comp, prot_bat)


# speedup vs baseline: 1.0017x; 1.0017x over previous
import functools

import jax
import jax.numpy as jnp
from jax import lax
from jax.experimental import pallas as pl
from jax.experimental.pallas import tpu as pltpu
from jax.experimental.pallas import tpu_sc as plsc

N = 10000
E = 320000
D = 128
B = 64
NC, NS = 2, 16
NW = NC * NS
NPAD = 10240
CHUNK = 128
E_PAD = 327680
EXTRA = E_PAD - E
TCHK = E_PAD // (NW * CHUNK)
PIPE = 2
RPT = NPAD // NS
ZR = 16

_mesh = plsc.VectorSubcoreMesh(
    core_axis_name="c", subcore_axis_name="s", num_cores=NC, num_subcores=NS)


@functools.partial(
    pl.kernel,
    out_type=jax.ShapeDtypeStruct((NC, 2, NPAD), jnp.float32),
    mesh=_mesh,
    scratch_types=[
        pltpu.VMEM_SHARED((NPAD,), jnp.float32),
        pltpu.VMEM_SHARED((NPAD,), jnp.float32),
        pltpu.VMEM((CHUNK,), jnp.int32),
        pltpu.VMEM((CHUNK,), jnp.int32),
        pltpu.VMEM((CHUNK,), jnp.int32),
        pltpu.VMEM((CHUNK,), jnp.int32),
        pltpu.VMEM((CHUNK,), jnp.float32),
        pltpu.VMEM((RPT,), jnp.float32),
        pltpu.SemaphoreType.DMA,
        pltpu.SemaphoreType.DMA,
        pltpu.SemaphoreType.DMA,
        pltpu.SemaphoreType.DMA,
    ],
)
def _deg_kernel(src_hbm, dst_hbm, out_hbm, do_sp, di_sp, is0, is1, id0, id1,
                ones_v, zb_v, il0, il1, sc0, sc1):
    isb = (is0, is1)
    idb = (id0, id1)
    ilsems = (il0, il1)
    scsems = (sc0, sc1)
    c = lax.axis_index("c")
    s = lax.axis_index("s")
    w = c * NS + s

    def _fill(i, _):
        zb_v[pl.ds(i * 16, 16)] = jnp.zeros((16,), jnp.float32)
        return 0
    lax.fori_loop(0, RPT // 16, _fill, 0)

    def _fill1(i, _):
        ones_v[pl.ds(i * 16, 16)] = jnp.ones((16,), jnp.float32)
        return 0
    lax.fori_loop(0, CHUNK // 16, _fill1, 0)

    pltpu.sync_copy(zb_v, do_sp.at[pl.ds(s * RPT, RPT)])
    pltpu.sync_copy(zb_v, di_sp.at[pl.ds(s * RPT, RPT)])
    plsc.subcore_barrier()

    def _grp(t, _):
        for k in range(PIPE):
            @pl.when(t > 0)
            def _():
                pltpu.make_async_copy(
                    ones_v, do_sp.at[isb[k]], scsems[k]).wait()
                pltpu.make_async_copy(
                    ones_v, di_sp.at[idb[k]], scsems[k]).wait()
            row = w * TCHK + PIPE * t + k
            pltpu.async_copy(src_hbm.at[row], isb[k], ilsems[k])
            pltpu.async_copy(dst_hbm.at[row], idb[k], ilsems[k])
        for k in range(PIPE):
            row = w * TCHK + PIPE * t + k
            pltpu.make_async_copy(src_hbm.at[row], isb[k], ilsems[k]).wait()
            pltpu.make_async_copy(dst_hbm.at[row], idb[k], ilsems[k]).wait()
            pltpu.async_copy(ones_v, do_sp.at[isb[k]], scsems[k], add=True)
            pltpu.async_copy(ones_v, di_sp.at[idb[k]], scsems[k], add=True)
        return 0
    lax.fori_loop(0, TCHK // PIPE, _grp, 0)
    for k in range(PIPE):
        pltpu.make_async_copy(ones_v, do_sp.at[isb[k]], scsems[k]).wait()
        pltpu.make_async_copy(ones_v, di_sp.at[idb[k]], scsems[k]).wait()
    plsc.subcore_barrier()

    pltpu.sync_copy(do_sp.at[pl.ds(s * RPT, RPT)], zb_v)
    pltpu.sync_copy(zb_v, out_hbm.at[c, 0, pl.ds(s * RPT, RPT)])
    pltpu.sync_copy(di_sp.at[pl.ds(s * RPT, RPT)], zb_v)
    pltpu.sync_copy(zb_v, out_hbm.at[c, 1, pl.ds(s * RPT, RPT)])


@functools.partial(
    pl.kernel,
    out_type=jax.ShapeDtypeStruct((NC, NPAD, D), jnp.float32),
    mesh=_mesh,
    scratch_types=[
        pltpu.VMEM_SHARED((NPAD, D), jnp.float32),
        pltpu.VMEM((2, CHUNK), jnp.int32),
        pltpu.VMEM((2, CHUNK), jnp.int32),
        pltpu.VMEM((CHUNK,), jnp.int32),
        pltpu.VMEM((CHUNK,), jnp.int32),
        pltpu.VMEM((PIPE, CHUNK, D), jnp.float32),
        pltpu.VMEM((ZR, D), jnp.float32),
        pltpu.SemaphoreType.DMA,
        pltpu.SemaphoreType.DMA,
        pltpu.SemaphoreType.DMA,
        pltpu.SemaphoreType.DMA,
    ],
)
def _agg_kernel(hs_hbm, idx2_hbm, out_hbm, acc_sp, ib0, ib1, id0, id1,
                rows_v, zb_v, g0, g1, s0, s1):
    gsems = (g0, g1)
    ssems = (s0, s1)
    ibb = (ib0, ib1)
    idb = (id0, id1)
    c = lax.axis_index("c")
    s = lax.axis_index("s")
    w = c * NS + s

    def _zrow(i, _):
        for k in range(D // 16):
            zb_v[i, pl.ds(16 * k, 16)] = jnp.zeros((16,), jnp.float32)
        return 0
    lax.fori_loop(0, ZR, _zrow, 0)

    def _zacc(i, _):
        pltpu.sync_copy(zb_v, acc_sp.at[pl.ds(s * RPT + ZR * i, ZR)])
        return 0
    lax.fori_loop(0, RPT // ZR, _zacc, 0)
    plsc.subcore_barrier()

    def _grp(t, _):
        for k in range(PIPE):
            row = w * TCHK + PIPE * t + k

            @pl.when(t > 0)
            def _():
                pltpu.make_async_copy(
                    rows_v.at[k], acc_sp.at[idb[k]], ssems[k]).wait()
            pltpu.sync_copy(idx2_hbm.at[row], ibb[k])
            pltpu.async_copy(hs_hbm.at[ibb[k].at[0]], rows_v.at[k], gsems[k])
            for m in range(CHUNK // 16):
                idb[k][pl.ds(16 * m, 16)] = ibb[k][1, pl.ds(16 * m, 16)]
        for k in range(PIPE):
            pltpu.make_async_copy(
                hs_hbm.at[ibb[k].at[0]], rows_v.at[k], gsems[k]).wait()
            pltpu.async_copy(
                rows_v.at[k], acc_sp.at[idb[k]], ssems[k], add=True)
        return 0
    lax.fori_loop(0, TCHK // PIPE, _grp, 0)
    for k in range(PIPE):
        pltpu.make_async_copy(
            rows_v.at[k], acc_sp.at[idb[k]], ssems[k]).wait()
    plsc.subcore_barrier()

    pltpu.sync_copy(acc_sp.at[pl.ds(s * RPT, RPT)],
                    out_hbm.at[c, pl.ds(s * RPT, RPT)])


BR = 512
NBLK = NPAD // BR


def _tc1_body(x_ref, w_ref, d0_ref, d1_ref, o_ref):
    inv = lax.rsqrt(jnp.maximum(d0_ref[...] + d1_ref[...], 1.0))
    h = jnp.dot(x_ref[...], w_ref[...], preferred_element_type=jnp.float32)
    o_ref[...] = h * inv[:, None]


def _tc1(x, W, do0, do1):
    return pl.pallas_call(
        _tc1_body,
        grid=(NBLK,),
        in_specs=[
            pl.BlockSpec((BR, D), lambda i: (i, 0)),
            pl.BlockSpec((D, D), lambda i: (0, 0)),
            pl.BlockSpec((BR,), lambda i: (i,)),
            pl.BlockSpec((BR,), lambda i: (i,)),
        ],
        out_specs=pl.BlockSpec((BR, D), lambda i: (i, 0)),
        out_shape=jax.ShapeDtypeStruct((NPAD, D), jnp.float32),
    )(x, W, do0, do1)


def _tc2_body(a0_ref, a1_ref, di0_ref, di1_ref, do0_ref, do1_ref,
              b_ref, g_ref, be_ref, w_ref, o_ref):
    agg = a0_ref[...] + a1_ref[...]
    inv_in = lax.rsqrt(jnp.maximum(di0_ref[...] + di1_ref[...], 1.0))
    out1 = g_ref[...] * (agg * inv_in[:, None] + b_ref[...]) + be_ref[...]
    out1 = jnp.maximum(out1, 0.0)
    inv_out = lax.rsqrt(jnp.maximum(do0_ref[...] + do1_ref[...], 1.0))
    h = jnp.dot(out1, w_ref[...], preferred_element_type=jnp.float32)
    o_ref[...] = h * inv_out[:, None]


def _tc2(a0, a1, di0, di1, do0, do1, b, g, be, W):
    return pl.pallas_call(
        _tc2_body,
        grid=(NBLK,),
        in_specs=[
            pl.BlockSpec((BR, D), lambda i: (i, 0)),
            pl.BlockSpec((BR, D), lambda i: (i, 0)),
            pl.BlockSpec((BR,), lambda i: (i,)),
            pl.BlockSpec((BR,), lambda i: (i,)),
            pl.BlockSpec((BR,), lambda i: (i,)),
            pl.BlockSpec((BR,), lambda i: (i,)),
            pl.BlockSpec((D,), lambda i: (0,)),
            pl.BlockSpec((D,), lambda i: (0,)),
            pl.BlockSpec((D,), lambda i: (0,)),
            pl.BlockSpec((D, D), lambda i: (0, 0)),
        ],
        out_specs=pl.BlockSpec((BR, D), lambda i: (i, 0)),
        out_shape=jax.ShapeDtypeStruct((NPAD, D), jnp.float32),
    )(a0, a1, di0, di1, do0, do1, b, g, be, W)


def _tc3_body(a0_ref, a1_ref, di0_ref, di1_ref, gid_ref, b_ref, g_ref,
              be_ref, wf1_ref, bf1_ref, wf2_ref, bf2_ref, prot_ref,
              wo1_ref, bo1_ref, wo2_ref, bo2_ref, wo3_ref, bo3_ref,
              z_ref, comp_ref, acc_ref):
    i = pl.program_id(0)

    @pl.when(i == 0)
    def _():
        acc_ref[...] = jnp.full((B, D), -jnp.inf, jnp.float32)

    agg = a0_ref[...] + a1_ref[...]
    inv_in = lax.rsqrt(jnp.maximum(di0_ref[...] + di1_ref[...], 1.0))
    f = g_ref[...] * (agg * inv_in[:, None] + b_ref[...]) + be_ref[...]
    f = jnp.maximum(f, 0.0)

    rows = i * BR + lax.broadcasted_iota(jnp.int32, (BR,), 0)
    valid = rows < N
    gid = gid_ref[...]
    g_lo = jnp.min(jnp.where(valid, gid, B - 1))
    g_hi = jnp.max(jnp.where(valid, gid, 0))

    def _upd(b, _):
        m = (gid == b) & valid
        p = jnp.where(m, 0.0, -jnp.inf)
        v = jnp.max(f + p[:, None], axis=0)
        cur = acc_ref[pl.ds(b, 1), :]
        acc_ref[pl.ds(b, 1), :] = jnp.maximum(cur, v[None, :])
        return 0
    lax.fori_loop(g_lo, g_hi + 1, _upd, 0)

    @pl.when(i == pl.num_programs(0) - 1)
    def _():
        pooled = acc_ref[...]
        h = jnp.dot(pooled, wf1_ref[...], preferred_element_type=jnp.float32)
        h = jnp.maximum(h + bf1_ref[...], 0.0)
        comp = jnp.dot(h, wf2_ref[...],
                       preferred_element_type=jnp.float32) + bf2_ref[...]
        comp_ref[...] = comp
        h2 = (jnp.dot(comp, wo1_ref[0:D, :], preferred_element_type=jnp.float32)
              + jnp.dot(prot_ref[...], wo1_ref[D:2 * D, :],
                        preferred_element_type=jnp.float32) + bo1_ref[...])
        h2 = jnp.maximum(h2, 0.0)
        h3 = jnp.maximum(
            jnp.dot(h2, wo2_ref[...], preferred_element_type=jnp.float32)
            + bo2_ref[...], 0.0)
        z_ref[...] = jnp.dot(
            h3, wo3_ref[...], preferred_element_type=jnp.float32) + bo3_ref[...]


def _tc3(a0, a1, di0, di1, gid, b, g, be, Wf1, bf1, Wf2, bf2, prot,
         Wo1, bo1, Wo2, bo2, Wo3, bo3):
    H2 = 2 * 128
    return pl.pallas_call(
        _tc3_body,
        grid=(NBLK,),
        in_specs=[
            pl.BlockSpec((BR, D), lambda i: (i, 0)),
            pl.BlockSpec((BR, D), lambda i: (i, 0)),
            pl.BlockSpec((BR,), lambda i: (i,)),
            pl.BlockSpec((BR,), lambda i: (i,)),
            pl.BlockSpec((BR,), lambda i: (i,)),
            pl.BlockSpec((D,), lambda i: (0,)),
            pl.BlockSpec((D,), lambda i: (0,)),
            pl.BlockSpec((D,), lambda i: (0,)),
            pl.BlockSpec((D, H2), lambda i: (0, 0)),
            pl.BlockSpec((H2,), lambda i: (0,)),
            pl.BlockSpec((H2, D), lambda i: (0, 0)),
            pl.BlockSpec((D,), lambda i: (0,)),
            pl.BlockSpec((B, D), lambda i: (0, 0)),
            pl.BlockSpec((2 * D, H2), lambda i: (0, 0)),
            pl.BlockSpec((H2,), lambda i: (0,)),
            pl.BlockSpec((H2, D), lambda i: (0, 0)),
            pl.BlockSpec((D,), lambda i: (0,)),
            pl.BlockSpec((D, 2), lambda i: (0, 0)),
            pl.BlockSpec((2,), lambda i: (0,)),
        ],
        out_specs=[
            pl.BlockSpec((B, 2), lambda i: (0, 0)),
            pl.BlockSpec((B, D), lambda i: (0, 0)),
        ],
        out_shape=[
            jax.ShapeDtypeStruct((B, 2), jnp.float32),
            jax.ShapeDtypeStruct((B, D), jnp.float32),
        ],
        scratch_shapes=[pltpu.VMEM((B, D), jnp.float32)],
    )(a0, a1, di0, di1, gid, b, g, be, Wf1, bf1, Wf2, bf2, prot,
      Wo1, bo1, Wo2, bo2, Wo3, bo3)


def kernel(x, edge_index, graph_ids, prot_bat, prot_list,
           W_g1, b_g1, gamma1, beta1, W_g2, b_g2, gamma2, beta2,
           W_fc1g, b_fc1g, W_fc2g, b_fc2g,
           W_o1, b_o1, W_o2, b_o2, W_o3, b_o3):
    pad_idx = N + jnp.arange(EXTRA, dtype=jnp.int32) % (NPAD - N)
    src2 = jnp.concatenate([edge_index[0], pad_idx]).reshape(-1, CHUNK)
    dst2 = jnp.concatenate([edge_index[1], pad_idx]).reshape(-1, CHUNK)
    idx2 = jnp.stack([src2, dst2], axis=1)
    x_p = jnp.pad(x, ((0, NPAD - N), (0, 0)))

    degs = _deg_kernel(src2, dst2)
    do0, do1 = degs[0, 0], degs[1, 0]
    di0, di1 = degs[0, 1], degs[1, 1]

    hs1 = _tc1(x_p, W_g1, do0, do1)
    agg1 = _agg_kernel(hs1, idx2)
    hs2 = _tc2(agg1[0], agg1[1], di0, di1, do0, do1,
               b_g1, gamma1, beta1, W_g2)
    agg2 = _agg_kernel(hs2, idx2)
    z, comp = _tc3(agg2[0], agg2[1], di0, di1, graph_ids,
                   b_g2, gamma2, beta2,
                   W_fc1g, b_fc1g, W_fc2g, b_fc2g, prot_bat,
                   W_o1, b_o1, W_o2, b_o2, W_o3, b_o3)
    return (z, comp, prot_bat)
